# Initial kernel scaffold; baseline (speedup 1.0000x reference)
import jax
import jax.numpy as jnp
from jax.experimental import pallas as pl


def kernel(x, ei, W1, b1, W2, b2):
    raise NotImplementedError("write your pallas kernel here")


# speedup vs baseline: 12.5130x; 12.5130x over previous
import functools

import jax
import jax.numpy as jnp
from jax import lax
from jax.experimental import pallas as pl
from jax.experimental.pallas import tpu as pltpu
from jax.experimental.pallas import tpu_sc as plsc

N = 10000
NPAD = 10240
E = 320000
NC = 2
NS = 16
NT = NC * NS
EPT = E // NT
K = 80
STEPS = EPT // K
RPT = NPAD // NS


def _edge_pass(width):
    mesh = plsc.VectorSubcoreMesh(core_axis_name="c", subcore_axis_name="s")

    @functools.partial(
        pl.kernel,
        mesh=mesh,
        out_type=jax.ShapeDtypeStruct((NC, NPAD, width), jnp.float32),
        scratch_types=[
            pltpu.VMEM((K,), jnp.int32),
            pltpu.VMEM((K,), jnp.int32),
            pltpu.VMEM((K, width), jnp.float32),
            pltpu.VMEM_SHARED((NPAD, width), jnp.float32),
            pltpu.SemaphoreType.DMA,
        ],
    )
    def k(g_hbm, src_hbm, dst_hbm, zeros_hbm, out_hbm, src_v, dst_v, rows_v,
          acc, sem):
        c = lax.axis_index("c")
        s = lax.axis_index("s")
        tid = c * NS + s
        pltpu.sync_copy(zeros_hbm, acc.at[pl.ds(s * RPT, RPT)])
        plsc.subcore_barrier()

        ebase = tid * EPT

        def body(i, carry):
            base = pl.multiple_of(ebase + i * K, 8)
            pltpu.sync_copy(src_hbm.at[pl.ds(base, K)], src_v)
            pltpu.sync_copy(dst_hbm.at[pl.ds(base, K)], dst_v)
            pltpu.async_copy(g_hbm.at[src_v], rows_v, sem).wait()
            pltpu.sync_copy(rows_v, acc.at[dst_v], add=True)
            return carry

        lax.fori_loop(0, STEPS, body, 0)
        plsc.subcore_barrier()
        pltpu.sync_copy(acc.at[pl.ds(s * RPT, RPT)],
                        out_hbm.at[c, pl.ds(s * RPT, RPT)])

    return k


DW = 128


def _deg_pass():
    mesh = plsc.VectorSubcoreMesh(core_axis_name="c", subcore_axis_name="s")

    @functools.partial(
        pl.kernel,
        mesh=mesh,
        out_type=jax.ShapeDtypeStruct((NC, NPAD, DW), jnp.float32),
        scratch_types=[
            pltpu.VMEM((K,), jnp.int32),
            pltpu.VMEM((K, DW), jnp.float32),
            pltpu.VMEM_SHARED((NPAD, DW), jnp.float32),
        ],
    )
    def k(dst_hbm, ones_hbm, zeros_hbm, out_hbm, dst_v, ones_v, acc):
        c = lax.axis_index("c")
        s = lax.axis_index("s")
        tid = c * NS + s
        pltpu.sync_copy(zeros_hbm, acc.at[pl.ds(s * RPT, RPT)])
        pltpu.sync_copy(ones_hbm, ones_v)
        plsc.subcore_barrier()

        ebase = tid * EPT

        def body(i, carry):
            base = pl.multiple_of(ebase + i * K, 8)
            pltpu.sync_copy(dst_hbm.at[pl.ds(base, K)], dst_v)
            pltpu.sync_copy(ones_v, acc.at[dst_v], add=True)
            return carry

        lax.fori_loop(0, STEPS, body, 0)
        plsc.subcore_barrier()
        pltpu.sync_copy(acc.at[pl.ds(s * RPT, RPT)],
                        out_hbm.at[c, pl.ds(s * RPT, RPT)])

    return k


def _dinv_from(dp_ref):
    deg = dp_ref[0, :, 0] + dp_ref[1, :, 0] + 1.0
    return lax.rsqrt(deg)[:, None]


BS = 1024
GRID = NPAD // BS


def _stage1(x, W1, dp):
    def body(x_ref, w_ref, dp_ref, g_ref):
        dinv = _dinv_from(dp_ref)
        h = jnp.dot(x_ref[...], w_ref[...], preferred_element_type=jnp.float32)
        g_ref[...] = dinv * h

    return pl.pallas_call(
        body,
        grid=(GRID,),
        in_specs=[
            pl.BlockSpec((BS, 128), lambda i: (i, 0)),
            pl.BlockSpec((128, 128), lambda i: (0, 0)),
            pl.BlockSpec((2, BS, 128), lambda i: (0, i, 0)),
        ],
        out_specs=pl.BlockSpec((BS, 128), lambda i: (i, 0)),
        out_shape=jax.ShapeDtypeStruct((NPAD, 128), jnp.float32),
    )(x, W1, dp)


def _stage2(s1, g1, dp, b1, W2p):
    def body(s_ref, g_ref, dp_ref, b_ref, w_ref, o_ref):
        dinv = _dinv_from(dp_ref)
        pre = dinv * (s_ref[0] + s_ref[1] + g_ref[...]) + b_ref[...][None, :]
        a = jnp.maximum(pre, 0.0)
        o_ref[...] = dinv * jnp.dot(a, w_ref[...],
                                    preferred_element_type=jnp.float32)

    return pl.pallas_call(
        body,
        grid=(GRID,),
        in_specs=[
            pl.BlockSpec((2, BS, 128), lambda i: (0, i, 0)),
            pl.BlockSpec((BS, 128), lambda i: (i, 0)),
            pl.BlockSpec((2, BS, 128), lambda i: (0, i, 0)),
            pl.BlockSpec((128,), lambda i: (0,)),
            pl.BlockSpec((128, 128), lambda i: (0, 0)),
        ],
        out_specs=pl.BlockSpec((BS, 128), lambda i: (i, 0)),
        out_shape=jax.ShapeDtypeStruct((NPAD, 128), jnp.float32),
    )(s1, g1, dp, b1, W2p)


def _stage3(s2, g2, dp, b2p):
    def body(s_ref, g_ref, dp_ref, b_ref, o_ref):
        dinv = _dinv_from(dp_ref)
        pre = dinv * (s_ref[0] + s_ref[1] + g_ref[...]) + b_ref[...][None, :]
        o_ref[...] = jnp.maximum(pre[:, :64], 0.0)

    return pl.pallas_call(
        body,
        grid=(GRID,),
        in_specs=[
            pl.BlockSpec((2, BS, 128), lambda i: (0, i, 0)),
            pl.BlockSpec((BS, 128), lambda i: (i, 0)),
            pl.BlockSpec((2, BS, 128), lambda i: (0, i, 0)),
            pl.BlockSpec((128,), lambda i: (0,)),
        ],
        out_specs=pl.BlockSpec((BS, 64), lambda i: (i, 0)),
        out_shape=jax.ShapeDtypeStruct((NPAD, 64), jnp.float32),
    )(s2, g2, dp, b2p)


def kernel(x, ei, W1, b1, W2, b2):
    src = ei[0]
    dst = ei[1]
    xp = jnp.zeros((NPAD, 128), jnp.float32).at[:N].set(x)

    zeros128 = jnp.zeros((RPT, 128), jnp.float32)
    ones128 = jnp.ones((K, DW), jnp.float32)
    dp = _deg_pass()(dst, ones128, zeros128)

    W2p = jnp.zeros((128, 128), jnp.float32).at[:, :64].set(W2)
    b2p = jnp.zeros((128,), jnp.float32).at[:64].set(b2)

    g1 = _stage1(xp, W1, dp)
    s1 = _edge_pass(128)(g1, src, dst, zeros128)
    g2 = _stage2(s1, g1, dp, b1, W2p)
    s2 = _edge_pass(128)(g2, src, dst, zeros128)
    out = _stage3(s2, g2, dp, b2p)
    return out[:N]


# speedup vs baseline: 19.8019x; 1.5825x over previous
import functools

import jax
import jax.numpy as jnp
from jax import lax
from jax.experimental import pallas as pl
from jax.experimental.pallas import tpu as pltpu
from jax.experimental.pallas import tpu_sc as plsc

N = 10000
NPAD = 10240
E = 320000
NC = 2
NS = 16
NT = NC * NS
EPT = E // NT
K = 80
STEPS = EPT // K
HALF = (STEPS - 1) // 2
RPT = NPAD // NS
DW = 128


def _edge_pass(width):
    mesh = plsc.VectorSubcoreMesh(core_axis_name="c", subcore_axis_name="s")

    @functools.partial(
        pl.kernel,
        mesh=mesh,
        out_type=jax.ShapeDtypeStruct((NC, NPAD, width), jnp.float32),
        scratch_types=[
            pltpu.VMEM((K,), jnp.int32),
            pltpu.VMEM((K,), jnp.int32),
            pltpu.VMEM((K,), jnp.int32),
            pltpu.VMEM((K,), jnp.int32),
            pltpu.VMEM((K, width), jnp.float32),
            pltpu.VMEM((K, width), jnp.float32),
            pltpu.VMEM_SHARED((NPAD, width), jnp.float32),
            pltpu.SemaphoreType.DMA,
            pltpu.SemaphoreType.DMA,
            pltpu.SemaphoreType.DMA,
            pltpu.SemaphoreType.DMA,
        ],
    )
    def k(g_hbm, src_hbm, dst_hbm, zeros_hbm, out_hbm, sa, da, sb, db,
          rows_a, rows_b, acc, ga, gb, ia, ib):
        c = lax.axis_index("c")
        s = lax.axis_index("s")
        tid = c * NS + s
        ebase = pl.multiple_of(tid * EPT, 16)
        pltpu.sync_copy(zeros_hbm, acc.at[pl.ds(s * RPT, RPT)])
        plsc.subcore_barrier()

        def eslice(i):
            return pl.ds(pl.multiple_of(ebase + i * K, 16), K)

        def load_idx(i, sv, dv, sem):
            pltpu.async_copy(src_hbm.at[eslice(i)], sv, sem)
            pltpu.async_copy(dst_hbm.at[eslice(i)], dv, sem)

        def wait_idx(i, sv, dv, sem):
            pltpu.make_async_copy(src_hbm.at[eslice(i)], sv, sem).wait()
            pltpu.make_async_copy(dst_hbm.at[eslice(i)], dv, sem).wait()

        pltpu.sync_copy(src_hbm.at[eslice(0)], sa)
        pltpu.sync_copy(dst_hbm.at[eslice(0)], da)
        pltpu.async_copy(g_hbm.at[sa], rows_a, ga)
        load_idx(1, sb, db, ib)

        def body(j, carry):
            a = j * 2
            b = a + 1
            pltpu.make_async_copy(g_hbm.at[sa], rows_a, ga).wait()
            wait_idx(b, sb, db, ib)
            pltpu.async_copy(g_hbm.at[sb], rows_b, gb)
            pltpu.sync_copy(rows_a, acc.at[da], add=True)

            @pl.when(j < HALF - 1)
            def _():
                load_idx(a + 2, sa, da, ia)

            pltpu.make_async_copy(g_hbm.at[sb], rows_b, gb).wait()

            @pl.when(j < HALF - 1)
            def _():
                wait_idx(a + 2, sa, da, ia)
                pltpu.async_copy(g_hbm.at[sa], rows_a, ga)

            pltpu.sync_copy(rows_b, acc.at[db], add=True)

            @pl.when(j < HALF - 1)
            def _():
                load_idx(b + 2, sb, db, ib)

            return carry

        lax.fori_loop(0, HALF, body, 0)

        t = STEPS - 1
        pltpu.sync_copy(src_hbm.at[eslice(t)], sa)
        pltpu.sync_copy(dst_hbm.at[eslice(t)], da)
        pltpu.async_copy(g_hbm.at[sa], rows_a, ga).wait()
        pltpu.sync_copy(rows_a, acc.at[da], add=True)

        plsc.subcore_barrier()
        pltpu.sync_copy(acc.at[pl.ds(s * RPT, RPT)],
                        out_hbm.at[c, pl.ds(s * RPT, RPT)])

    return k


def _deg_pass():
    mesh = plsc.VectorSubcoreMesh(core_axis_name="c", subcore_axis_name="s")

    @functools.partial(
        pl.kernel,
        mesh=mesh,
        out_type=jax.ShapeDtypeStruct((NC, NPAD, DW), jnp.float32),
        scratch_types=[
            pltpu.VMEM((K,), jnp.int32),
            pltpu.VMEM((K, DW), jnp.float32),
            pltpu.VMEM_SHARED((NPAD, DW), jnp.float32),
        ],
    )
    def k(dst_hbm, ones_hbm, zeros_hbm, out_hbm, dst_v, ones_v, acc):
        c = lax.axis_index("c")
        s = lax.axis_index("s")
        tid = c * NS + s
        ebase = pl.multiple_of(tid * EPT, 16)
        pltpu.sync_copy(zeros_hbm, acc.at[pl.ds(s * RPT, RPT)])
        pltpu.sync_copy(ones_hbm, ones_v)
        plsc.subcore_barrier()

        def body(i, carry):
            base = pl.multiple_of(ebase + i * K, 16)
            pltpu.sync_copy(dst_hbm.at[pl.ds(base, K)], dst_v)
            pltpu.sync_copy(ones_v, acc.at[dst_v], add=True)
            return carry

        lax.fori_loop(0, STEPS, body, 0)
        plsc.subcore_barrier()
        pltpu.sync_copy(acc.at[pl.ds(s * RPT, RPT)],
                        out_hbm.at[c, pl.ds(s * RPT, RPT)])

    return k


def _dinv_from(dp_ref):
    deg = dp_ref[0, :, 0] + dp_ref[1, :, 0] + 1.0
    return lax.rsqrt(deg)[:, None]


BS = 1024
GRID = NPAD // BS


def _mm1(x, W1):
    def body(x_ref, w_ref, h_ref):
        h_ref[...] = jnp.dot(x_ref[...], w_ref[...],
                             preferred_element_type=jnp.float32)

    return pl.pallas_call(
        body,
        grid=(GRID,),
        in_specs=[
            pl.BlockSpec((BS, 128), lambda i: (i, 0)),
            pl.BlockSpec((128, 128), lambda i: (0, 0)),
        ],
        out_specs=pl.BlockSpec((BS, 128), lambda i: (i, 0)),
        out_shape=jax.ShapeDtypeStruct((NPAD, 128), jnp.float32),
    )(x, W1)


def _scale1(h1, dp):
    def body(h_ref, dp_ref, g_ref):
        g_ref[...] = _dinv_from(dp_ref) * h_ref[...]

    return pl.pallas_call(
        body,
        grid=(GRID,),
        in_specs=[
            pl.BlockSpec((BS, 128), lambda i: (i, 0)),
            pl.BlockSpec((2, BS, DW), lambda i: (0, i, 0)),
        ],
        out_specs=pl.BlockSpec((BS, 128), lambda i: (i, 0)),
        out_shape=jax.ShapeDtypeStruct((NPAD, 128), jnp.float32),
    )(h1, dp)


def _stage2(s1, g1, dp, b1, W2p):
    def body(s_ref, g_ref, dp_ref, b_ref, w_ref, o_ref):
        dinv = _dinv_from(dp_ref)
        pre = dinv * (s_ref[0] + s_ref[1] + g_ref[...]) + b_ref[...][None, :]
        a = jnp.maximum(pre, 0.0)
        o_ref[...] = dinv * jnp.dot(a, w_ref[...],
                                    preferred_element_type=jnp.float32)

    return pl.pallas_call(
        body,
        grid=(GRID,),
        in_specs=[
            pl.BlockSpec((2, BS, 128), lambda i: (0, i, 0)),
            pl.BlockSpec((BS, 128), lambda i: (i, 0)),
            pl.BlockSpec((2, BS, DW), lambda i: (0, i, 0)),
            pl.BlockSpec((128,), lambda i: (0,)),
            pl.BlockSpec((128, 128), lambda i: (0, 0)),
        ],
        out_specs=pl.BlockSpec((BS, 128), lambda i: (i, 0)),
        out_shape=jax.ShapeDtypeStruct((NPAD, 128), jnp.float32),
    )(s1, g1, dp, b1, W2p)


def _stage3(s2, g2, dp, b2p):
    def body(s_ref, g_ref, dp_ref, b_ref, o_ref):
        dinv = _dinv_from(dp_ref)
        pre = dinv * (s_ref[0] + s_ref[1] + g_ref[...]) + b_ref[...][None, :]
        o_ref[...] = jnp.maximum(pre[:, :64], 0.0)

    return pl.pallas_call(
        body,
        grid=(GRID,),
        in_specs=[
            pl.BlockSpec((2, BS, 128), lambda i: (0, i, 0)),
            pl.BlockSpec((BS, 128), lambda i: (i, 0)),
            pl.BlockSpec((2, BS, DW), lambda i: (0, i, 0)),
            pl.BlockSpec((128,), lambda i: (0,)),
        ],
        out_specs=pl.BlockSpec((BS, 64), lambda i: (i, 0)),
        out_shape=jax.ShapeDtypeStruct((NPAD, 64), jnp.float32),
    )(s2, g2, dp, b2p)


def kernel(x, ei, W1, b1, W2, b2):
    src = ei[0]
    dst = ei[1]
    xp = jnp.zeros((NPAD, 128), jnp.float32).at[:N].set(x)

    zeros128 = jnp.zeros((RPT, 128), jnp.float32)
    ones_dw = jnp.ones((K, DW), jnp.float32)
    dp = _deg_pass()(dst, ones_dw, zeros128)

    W2p = jnp.zeros((128, 128), jnp.float32).at[:, :64].set(W2)
    b2p = jnp.zeros((128,), jnp.float32).at[:64].set(b2)

    h1 = _mm1(xp, W1)
    g1 = _scale1(h1, dp)
    s1 = _edge_pass(128)(g1, src, dst, zeros128)
    g2 = _stage2(s1, g1, dp, b1, W2p)
    s2 = _edge_pass(128)(g2, src, dst, zeros128)
    out = _stage3(s2, g2, dp, b2p)
    return out[:N]


# speedup vs baseline: 28.6911x; 1.4489x over previous
import functools

import jax
import jax.numpy as jnp
from jax import lax
from jax.experimental import pallas as pl
from jax.experimental.pallas import tpu as pltpu
from jax.experimental.pallas import tpu_sc as plsc

N = 10000
NPAD = 10240
E = 320000
NC = 2
NS = 16
NT = NC * NS
EPT = E // NT
K = 80
STEPS = EPT // K
HALF = (STEPS - 1) // 2
RPT = NPAD // NS
DW = 128


def _edge_pass(width):
    mesh = plsc.VectorSubcoreMesh(core_axis_name="c", subcore_axis_name="s")

    @functools.partial(
        pl.kernel,
        mesh=mesh,
        out_type=jax.ShapeDtypeStruct((NC, NPAD, width), jnp.float32),
        scratch_types=(
            [pltpu.VMEM((K,), jnp.int32)] * 8
            + [pltpu.VMEM((K, width), jnp.float32)] * 4
            + [pltpu.VMEM_SHARED((NPAD, width), jnp.float32)]
            + [pltpu.SemaphoreType.DMA] * 8
        ),
    )
    def k(g_hbm, src_hbm, dst_hbm, zeros_hbm, out_hbm,
          s0, d0, s1, d1, s2, d2, s3, d3, r0, r1, r2, r3, acc,
          g0, g1, g2, g3, i0, i1, i2, i3):
        c = lax.axis_index("c")
        s = lax.axis_index("s")
        tid = c * NS + s
        ebase = pl.multiple_of(tid * EPT, 16)
        pltpu.sync_copy(zeros_hbm, acc.at[pl.ds(s * RPT, RPT)])
        plsc.subcore_barrier()

        def eslice(i):
            return pl.ds(pl.multiple_of(ebase + i * K, 16), K)

        def load_idx(i, sv, dv, sem):
            pltpu.async_copy(src_hbm.at[eslice(i)], sv, sem)
            pltpu.async_copy(dst_hbm.at[eslice(i)], dv, sem)

        def wait_idx(i, sv, dv, sem):
            pltpu.make_async_copy(src_hbm.at[eslice(i)], sv, sem).wait()
            pltpu.make_async_copy(dst_hbm.at[eslice(i)], dv, sem).wait()

        lanes = [(s0, d0, r0, g0, i0), (s1, d1, r1, g1, i1),
                 (s2, d2, r2, g2, i2), (s3, d3, r3, g3, i3)]

        pltpu.sync_copy(src_hbm.at[eslice(0)], s0)
        pltpu.sync_copy(dst_hbm.at[eslice(0)], d0)
        pltpu.sync_copy(src_hbm.at[eslice(1)], s1)
        pltpu.sync_copy(dst_hbm.at[eslice(1)], d1)
        pltpu.async_copy(g_hbm.at[s0], r0, g0)
        pltpu.async_copy(g_hbm.at[s1], r1, g1)
        load_idx(2, s2, d2, i2)
        load_idx(3, s3, d3, i3)

        def step(j, LA, LB, LYA, LYB):
            a = j * 2
            b = a + 1
            sA, dA, rA, gA, iA = LA
            sB, dB, rB, gB, iB = LB
            sYA, dYA, rYA, gYA, iYA = LYA
            sYB, dYB, rYB, gYB, iYB = LYB

            pltpu.make_async_copy(g_hbm.at[sA], rA, gA).wait()

            @pl.when(j < HALF - 1)
            def _():
                wait_idx(a + 2, sYA, dYA, iYA)
                pltpu.async_copy(g_hbm.at[sYA], rYA, gYA)

            pltpu.sync_copy(rA, acc.at[dA], add=True)

            @pl.when(j < HALF - 2)
            def _():
                load_idx(a + 4, sA, dA, iA)

            pltpu.make_async_copy(g_hbm.at[sB], rB, gB).wait()

            @pl.when(j < HALF - 1)
            def _():
                wait_idx(b + 2, sYB, dYB, iYB)
                pltpu.async_copy(g_hbm.at[sYB], rYB, gYB)

            pltpu.sync_copy(rB, acc.at[dB], add=True)

            @pl.when(j < HALF - 2)
            def _():
                load_idx(b + 4, sB, dB, iB)

        def body(j, carry):
            @pl.when(lax.rem(j, 2) == 0)
            def _():
                step(j, lanes[0], lanes[1], lanes[2], lanes[3])

            @pl.when(lax.rem(j, 2) == 1)
            def _():
                step(j, lanes[2], lanes[3], lanes[0], lanes[1])

            return carry

        lax.fori_loop(0, HALF, body, 0)

        t = STEPS - 1
        pltpu.sync_copy(src_hbm.at[eslice(t)], s0)
        pltpu.sync_copy(dst_hbm.at[eslice(t)], d0)
        pltpu.async_copy(g_hbm.at[s0], r0, g0).wait()
        pltpu.sync_copy(r0, acc.at[d0], add=True)

        plsc.subcore_barrier()
        pltpu.sync_copy(acc.at[pl.ds(s * RPT, RPT)],
                        out_hbm.at[c, pl.ds(s * RPT, RPT)])

    return k


def _deg_pass():
    mesh = plsc.VectorSubcoreMesh(core_axis_name="c", subcore_axis_name="s")

    @functools.partial(
        pl.kernel,
        mesh=mesh,
        out_type=jax.ShapeDtypeStruct((NC, NPAD, DW), jnp.float32),
        scratch_types=[
            pltpu.VMEM((K,), jnp.int32),
            pltpu.VMEM((K,), jnp.int32),
            pltpu.VMEM((K, DW), jnp.float32),
            pltpu.VMEM_SHARED((NPAD, DW), jnp.float32),
            pltpu.SemaphoreType.DMA,
            pltpu.SemaphoreType.DMA,
        ],
    )
    def k(dst_hbm, ones_hbm, zeros_hbm, out_hbm, da, db, ones_v, acc, ia, ib):
        c = lax.axis_index("c")
        s = lax.axis_index("s")
        tid = c * NS + s
        ebase = pl.multiple_of(tid * EPT, 16)
        pltpu.sync_copy(zeros_hbm, acc.at[pl.ds(s * RPT, RPT)])
        pltpu.sync_copy(ones_hbm, ones_v)
        plsc.subcore_barrier()

        def eslice(i):
            return pl.ds(pl.multiple_of(ebase + i * K, 16), K)

        pltpu.async_copy(dst_hbm.at[eslice(0)], da, ia)
        pltpu.async_copy(dst_hbm.at[eslice(1)], db, ib)

        def body(j, carry):
            a = j * 2
            b = a + 1
            pltpu.make_async_copy(dst_hbm.at[eslice(a)], da, ia).wait()
            pltpu.sync_copy(ones_v, acc.at[da], add=True)

            @pl.when(j < HALF - 1)
            def _():
                pltpu.async_copy(dst_hbm.at[eslice(a + 2)], da, ia)

            pltpu.make_async_copy(dst_hbm.at[eslice(b)], db, ib).wait()
            pltpu.sync_copy(ones_v, acc.at[db], add=True)

            @pl.when(j < HALF - 1)
            def _():
                pltpu.async_copy(dst_hbm.at[eslice(b + 2)], db, ib)

            return carry

        lax.fori_loop(0, HALF, body, 0)
        t = STEPS - 1
        pltpu.sync_copy(dst_hbm.at[eslice(t)], da)
        pltpu.sync_copy(ones_v, acc.at[da], add=True)
        plsc.subcore_barrier()
        pltpu.sync_copy(acc.at[pl.ds(s * RPT, RPT)],
                        out_hbm.at[c, pl.ds(s * RPT, RPT)])

    return k


def _dinv_from(dp_ref):
    deg = dp_ref[0, :, 0] + dp_ref[1, :, 0] + 1.0
    return lax.rsqrt(deg)[:, None]


BS = 1024
GRID = NPAD // BS


def _mm1(x, W1):
    def body(x_ref, w_ref, h_ref):
        h_ref[...] = jnp.dot(x_ref[...], w_ref[...],
                             preferred_element_type=jnp.float32)

    return pl.pallas_call(
        body,
        grid=(GRID,),
        in_specs=[
            pl.BlockSpec((BS, 128), lambda i: (i, 0)),
            pl.BlockSpec((128, 128), lambda i: (0, 0)),
        ],
        out_specs=pl.BlockSpec((BS, 128), lambda i: (i, 0)),
        out_shape=jax.ShapeDtypeStruct((NPAD, 128), jnp.float32),
    )(x, W1)


def _scale1(h1, dp):
    def body(h_ref, dp_ref, g_ref):
        g_ref[...] = _dinv_from(dp_ref) * h_ref[...]

    return pl.pallas_call(
        body,
        grid=(GRID,),
        in_specs=[
            pl.BlockSpec((BS, 128), lambda i: (i, 0)),
            pl.BlockSpec((2, BS, DW), lambda i: (0, i, 0)),
        ],
        out_specs=pl.BlockSpec((BS, 128), lambda i: (i, 0)),
        out_shape=jax.ShapeDtypeStruct((NPAD, 128), jnp.float32),
    )(h1, dp)


def _stage2(s1, g1, dp, b1, W2p):
    def body(s_ref, g_ref, dp_ref, b_ref, w_ref, o_ref):
        dinv = _dinv_from(dp_ref)
        pre = dinv * (s_ref[0] + s_ref[1] + g_ref[...]) + b_ref[...][None, :]
        a = jnp.maximum(pre, 0.0)
        o_ref[...] = dinv * jnp.dot(a, w_ref[...],
                                    preferred_element_type=jnp.float32)

    return pl.pallas_call(
        body,
        grid=(GRID,),
        in_specs=[
            pl.BlockSpec((2, BS, 128), lambda i: (0, i, 0)),
            pl.BlockSpec((BS, 128), lambda i: (i, 0)),
            pl.BlockSpec((2, BS, DW), lambda i: (0, i, 0)),
            pl.BlockSpec((128,), lambda i: (0,)),
            pl.BlockSpec((128, 128), lambda i: (0, 0)),
        ],
        out_specs=pl.BlockSpec((BS, 128), lambda i: (i, 0)),
        out_shape=jax.ShapeDtypeStruct((NPAD, 128), jnp.float32),
    )(s1, g1, dp, b1, W2p)


def _stage3(s2, g2, dp, b2p):
    def body(s_ref, g_ref, dp_ref, b_ref, o_ref):
        dinv = _dinv_from(dp_ref)
        pre = dinv * (s_ref[0] + s_ref[1] + g_ref[...]) + b_ref[...][None, :]
        o_ref[...] = jnp.maximum(pre[:, :64], 0.0)

    return pl.pallas_call(
        body,
        grid=(GRID,),
        in_specs=[
            pl.BlockSpec((2, BS, 128), lambda i: (0, i, 0)),
            pl.BlockSpec((BS, 128), lambda i: (i, 0)),
            pl.BlockSpec((2, BS, DW), lambda i: (0, i, 0)),
            pl.BlockSpec((128,), lambda i: (0,)),
        ],
        out_specs=pl.BlockSpec((BS, 64), lambda i: (i, 0)),
        out_shape=jax.ShapeDtypeStruct((NPAD, 64), jnp.float32),
    )(s2, g2, dp, b2p)


def kernel(x, ei, W1, b1, W2, b2):
    src = ei[0]
    dst = ei[1]
    xp = jnp.zeros((NPAD, 128), jnp.float32).at[:N].set(x)

    zeros128 = jnp.zeros((RPT, 128), jnp.float32)
    ones_dw = jnp.ones((K, DW), jnp.float32)
    dp = _deg_pass()(dst, ones_dw, zeros128)

    W2p = jnp.zeros((128, 128), jnp.float32).at[:, :64].set(W2)
    b2p = jnp.zeros((128,), jnp.float32).at[:64].set(b2)

    h1 = _mm1(xp, W1)
    g1 = _scale1(h1, dp)
    s1 = _edge_pass(128)(g1, src, dst, zeros128)
    g2 = _stage2(s1, g1, dp, b1, W2p)
    s2 = _edge_pass(128)(g2, src, dst, zeros128)
    out = _stage3(s2, g2, dp, b2p)
    return out[:N]


# speedup vs baseline: 29.0914x; 1.0140x over previous
import functools

import jax
import jax.numpy as jnp
from jax import lax
from jax.experimental import pallas as pl
from jax.experimental.pallas import tpu as pltpu
from jax.experimental.pallas import tpu_sc as plsc

N = 10000
NPAD = 10240
E = 320000
NC = 2
NS = 16
NT = NC * NS
EPT = E // NT
K = 80
STEPS = EPT // K
HALF = (STEPS - 1) // 2
RPT = NPAD // NS
DW = 128


def _edge_pass(width):
    mesh = plsc.VectorSubcoreMesh(core_axis_name="c", subcore_axis_name="s")

    @functools.partial(
        pl.kernel,
        mesh=mesh,
        out_type=jax.ShapeDtypeStruct((NC, NPAD, width), jnp.float32),
        scratch_types=(
            [pltpu.VMEM((K,), jnp.int32)] * 8
            + [pltpu.VMEM((K, width), jnp.float32)] * 4
            + [pltpu.VMEM_SHARED((NPAD, width), jnp.float32)]
            + [pltpu.SemaphoreType.DMA] * 8
        ),
    )
    def k(g_hbm, src_hbm, dst_hbm, zeros_hbm, out_hbm,
          s0, d0, s1, d1, s2, d2, s3, d3, r0, r1, r2, r3, acc,
          g0, g1, g2, g3, i0, i1, i2, i3):
        c = lax.axis_index("c")
        s = lax.axis_index("s")
        tid = c * NS + s
        ebase = pl.multiple_of(tid * EPT, 16)
        pltpu.sync_copy(zeros_hbm, acc.at[pl.ds(s * RPT, RPT)])
        plsc.subcore_barrier()

        def eslice(i):
            return pl.ds(pl.multiple_of(ebase + i * K, 16), K)

        def load_idx(i, sv, dv, sem):
            pltpu.async_copy(src_hbm.at[eslice(i)], sv, sem)
            pltpu.async_copy(dst_hbm.at[eslice(i)], dv, sem)

        def wait_idx(i, sv, dv, sem):
            pltpu.make_async_copy(src_hbm.at[eslice(i)], sv, sem).wait()
            pltpu.make_async_copy(dst_hbm.at[eslice(i)], dv, sem).wait()

        lanes = [(s0, d0, r0, g0, i0), (s1, d1, r1, g1, i1),
                 (s2, d2, r2, g2, i2), (s3, d3, r3, g3, i3)]

        pltpu.sync_copy(src_hbm.at[eslice(0)], s0)
        pltpu.sync_copy(dst_hbm.at[eslice(0)], d0)
        pltpu.sync_copy(src_hbm.at[eslice(1)], s1)
        pltpu.sync_copy(dst_hbm.at[eslice(1)], d1)
        pltpu.async_copy(g_hbm.at[s0], r0, g0)
        pltpu.async_copy(g_hbm.at[s1], r1, g1)
        load_idx(2, s2, d2, i2)
        load_idx(3, s3, d3, i3)

        def step(j, LA, LB, LYA, LYB):
            a = j * 2
            b = a + 1
            sA, dA, rA, gA, iA = LA
            sB, dB, rB, gB, iB = LB
            sYA, dYA, rYA, gYA, iYA = LYA
            sYB, dYB, rYB, gYB, iYB = LYB

            pltpu.make_async_copy(g_hbm.at[sA], rA, gA).wait()

            @pl.when(j < HALF - 1)
            def _():
                wait_idx(a + 2, sYA, dYA, iYA)
                pltpu.async_copy(g_hbm.at[sYA], rYA, gYA)

            pltpu.sync_copy(rA, acc.at[dA], add=True)

            @pl.when(j < HALF - 2)
            def _():
                load_idx(a + 4, sA, dA, iA)

            pltpu.make_async_copy(g_hbm.at[sB], rB, gB).wait()

            @pl.when(j < HALF - 1)
            def _():
                wait_idx(b + 2, sYB, dYB, iYB)
                pltpu.async_copy(g_hbm.at[sYB], rYB, gYB)

            pltpu.sync_copy(rB, acc.at[dB], add=True)

            @pl.when(j < HALF - 2)
            def _():
                load_idx(b + 4, sB, dB, iB)

        def body(j, carry):
            @pl.when(lax.rem(j, 2) == 0)
            def _():
                step(j, lanes[0], lanes[1], lanes[2], lanes[3])

            @pl.when(lax.rem(j, 2) == 1)
            def _():
                step(j, lanes[2], lanes[3], lanes[0], lanes[1])

            return carry

        lax.fori_loop(0, HALF, body, 0)

        t = STEPS - 1
        pltpu.sync_copy(src_hbm.at[eslice(t)], s0)
        pltpu.sync_copy(dst_hbm.at[eslice(t)], d0)
        pltpu.async_copy(g_hbm.at[s0], r0, g0).wait()
        pltpu.sync_copy(r0, acc.at[d0], add=True)

        plsc.subcore_barrier()
        pltpu.sync_copy(acc.at[pl.ds(s * RPT, RPT)],
                        out_hbm.at[c, pl.ds(s * RPT, RPT)])

    return k


def _deg_pass():
    mesh = plsc.VectorSubcoreMesh(core_axis_name="c", subcore_axis_name="s")

    @functools.partial(
        pl.kernel,
        mesh=mesh,
        out_type=jax.ShapeDtypeStruct((NC, NPAD, DW), jnp.float32),
        scratch_types=[
            pltpu.VMEM((K,), jnp.int32),
            pltpu.VMEM((K,), jnp.int32),
            pltpu.VMEM((K, DW), jnp.float32),
            pltpu.VMEM_SHARED((NPAD, DW), jnp.float32),
            pltpu.SemaphoreType.DMA,
            pltpu.SemaphoreType.DMA,
        ],
    )
    def k(dst_hbm, ones_hbm, zeros_hbm, out_hbm, da, db, ones_v, acc, ia, ib):
        c = lax.axis_index("c")
        s = lax.axis_index("s")
        tid = c * NS + s
        ebase = pl.multiple_of(tid * EPT, 16)
        pltpu.sync_copy(zeros_hbm, acc.at[pl.ds(s * RPT, RPT)])
        pltpu.sync_copy(ones_hbm, ones_v)
        plsc.subcore_barrier()

        def eslice(i):
            return pl.ds(pl.multiple_of(ebase + i * K, 16), K)

        pltpu.async_copy(dst_hbm.at[eslice(0)], da, ia)
        pltpu.async_copy(dst_hbm.at[eslice(1)], db, ib)

        def body(j, carry):
            a = j * 2
            b = a + 1
            pltpu.make_async_copy(dst_hbm.at[eslice(a)], da, ia).wait()
            pltpu.sync_copy(ones_v, acc.at[da], add=True)

            @pl.when(j < HALF - 1)
            def _():
                pltpu.async_copy(dst_hbm.at[eslice(a + 2)], da, ia)

            pltpu.make_async_copy(dst_hbm.at[eslice(b)], db, ib).wait()
            pltpu.sync_copy(ones_v, acc.at[db], add=True)

            @pl.when(j < HALF - 1)
            def _():
                pltpu.async_copy(dst_hbm.at[eslice(b + 2)], db, ib)

            return carry

        lax.fori_loop(0, HALF, body, 0)
        t = STEPS - 1
        pltpu.sync_copy(dst_hbm.at[eslice(t)], da)
        pltpu.sync_copy(ones_v, acc.at[da], add=True)
        plsc.subcore_barrier()
        pltpu.sync_copy(acc.at[pl.ds(s * RPT, RPT)],
                        out_hbm.at[c, pl.ds(s * RPT, RPT)])

    return k


def _dinv_from(dp_ref):
    deg = dp_ref[0, :, 0] + dp_ref[1, :, 0] + 1.0
    return lax.rsqrt(deg)[:, None]


BS = 1024
GRID = NPAD // BS


def _stage1(x, W1, dp):
    def body(x_ref, w_ref, dp_ref, g_ref):
        h = jnp.dot(x_ref[...], w_ref[...], preferred_element_type=jnp.float32)
        g_ref[...] = _dinv_from(dp_ref) * h

    return pl.pallas_call(
        body,
        grid=(GRID,),
        in_specs=[
            pl.BlockSpec((BS, 128), lambda i: (i, 0)),
            pl.BlockSpec((128, 128), lambda i: (0, 0)),
            pl.BlockSpec((2, BS, DW), lambda i: (0, i, 0)),
        ],
        out_specs=pl.BlockSpec((BS, 128), lambda i: (i, 0)),
        out_shape=jax.ShapeDtypeStruct((NPAD, 128), jnp.float32),
    )(x, W1, dp)


def _stage2(s1, g1, dp, b1, W2p):
    def body(s_ref, g_ref, dp_ref, b_ref, w_ref, o_ref):
        dinv = _dinv_from(dp_ref)
        pre = dinv * (s_ref[0] + s_ref[1] + g_ref[...]) + b_ref[...][None, :]
        a = jnp.maximum(pre, 0.0)
        o_ref[...] = dinv * jnp.dot(a, w_ref[...],
                                    preferred_element_type=jnp.float32)

    return pl.pallas_call(
        body,
        grid=(GRID,),
        in_specs=[
            pl.BlockSpec((2, BS, 128), lambda i: (0, i, 0)),
            pl.BlockSpec((BS, 128), lambda i: (i, 0)),
            pl.BlockSpec((2, BS, DW), lambda i: (0, i, 0)),
            pl.BlockSpec((128,), lambda i: (0,)),
            pl.BlockSpec((128, 128), lambda i: (0, 0)),
        ],
        out_specs=pl.BlockSpec((BS, 128), lambda i: (i, 0)),
        out_shape=jax.ShapeDtypeStruct((NPAD, 128), jnp.float32),
    )(s1, g1, dp, b1, W2p)


def _stage3(s2, g2, dp, b2p):
    def body(s_ref, g_ref, dp_ref, b_ref, o_ref):
        dinv = _dinv_from(dp_ref)
        pre = dinv * (s_ref[0] + s_ref[1] + g_ref[...]) + b_ref[...][None, :]
        o_ref[...] = jnp.maximum(pre[:, :64], 0.0)

    return pl.pallas_call(
        body,
        grid=(GRID,),
        in_specs=[
            pl.BlockSpec((2, BS, 128), lambda i: (0, i, 0)),
            pl.BlockSpec((BS, 128), lambda i: (i, 0)),
            pl.BlockSpec((2, BS, DW), lambda i: (0, i, 0)),
            pl.BlockSpec((128,), lambda i: (0,)),
        ],
        out_specs=pl.BlockSpec((BS, 64), lambda i: (i, 0)),
        out_shape=jax.ShapeDtypeStruct((NPAD, 64), jnp.float32),
    )(s2, g2, dp, b2p)


def kernel(x, ei, W1, b1, W2, b2):
    src = ei[0]
    dst = ei[1]
    xp = jnp.zeros((NPAD, 128), jnp.float32).at[:N].set(x)

    zeros128 = jnp.zeros((RPT, 128), jnp.float32)
    ones_dw = jnp.ones((K, DW), jnp.float32)
    dp = _deg_pass()(dst, ones_dw, zeros128)

    W2p = jnp.zeros((128, 128), jnp.float32).at[:, :64].set(W2)
    b2p = jnp.zeros((128,), jnp.float32).at[:64].set(b2)

    g1 = _stage1(xp, W1, dp)
    s1 = _edge_pass(128)(g1, src, dst, zeros128)
    g2 = _stage2(s1, g1, dp, b1, W2p)
    s2 = _edge_pass(128)(g2, src, dst, zeros128)
    out = _stage3(s2, g2, dp, b2p)
    return out[:N]


# speedup vs baseline: 30.3016x; 1.0416x over previous
import functools

import jax
import jax.numpy as jnp
from jax import lax
from jax.experimental import pallas as pl
from jax.experimental.pallas import tpu as pltpu
from jax.experimental.pallas import tpu_sc as plsc

N = 10000
NPAD = 10240
E = 320000
NC = 2
NS = 16
NT = NC * NS
EPT = E // NT
K = 80
STEPS = EPT // K
HALF = (STEPS - 1) // 2
RPT = NPAD // NS
DW = 128


def _edge_pass(width):
    mesh = plsc.VectorSubcoreMesh(core_axis_name="c", subcore_axis_name="s")

    @functools.partial(
        pl.kernel,
        mesh=mesh,
        out_type=jax.ShapeDtypeStruct((NC, NPAD, width), jnp.float32),
        scratch_types=(
            [pltpu.VMEM((K,), jnp.int32)] * 16
            + [pltpu.VMEM((K, width), jnp.float32)] * 4
            + [pltpu.VMEM_SHARED((NPAD, width), jnp.float32)]
            + [pltpu.SemaphoreType.DMA] * 12
        ),
    )
    def k(g_hbm, src_hbm, dst_hbm, zeros_hbm, out_hbm,
          s0, d0, s1, d1, s2, d2, s3, d3, s4, d4, s5, d5, s6, d6, s7, d7,
          r0, r1, r2, r3, acc,
          g0, g1, g2, g3, i0, i1, i2, i3, i4, i5, i6, i7):
        c = lax.axis_index("c")
        s = lax.axis_index("s")
        tid = c * NS + s
        ebase = pl.multiple_of(tid * EPT, 16)
        pltpu.sync_copy(zeros_hbm, acc.at[pl.ds(s * RPT, RPT)])
        plsc.subcore_barrier()

        def eslice(i):
            return pl.ds(pl.multiple_of(ebase + i * K, 16), K)

        def load_idx(i, sv, dv, sem):
            pltpu.async_copy(src_hbm.at[eslice(i)], sv, sem)
            pltpu.async_copy(dst_hbm.at[eslice(i)], dv, sem)

        def wait_idx(i, sv, dv, sem):
            pltpu.make_async_copy(src_hbm.at[eslice(i)], sv, sem).wait()
            pltpu.make_async_copy(dst_hbm.at[eslice(i)], dv, sem).wait()

        idx = [(s0, d0, i0), (s1, d1, i1), (s2, d2, i2), (s3, d3, i3),
               (s4, d4, i4), (s5, d5, i5), (s6, d6, i6), (s7, d7, i7)]
        rows = [(r0, g0), (r1, g1), (r2, g2), (r3, g3)]

        for cch in range(8):
            sv, dv, sem = idx[cch]
            load_idx(cch, sv, dv, sem)
        for cch in range(3):
            sv, dv, sem = idx[cch]
            rv, gsem = rows[cch]
            wait_idx(cch, sv, dv, sem)
            pltpu.async_copy(g_hbm.at[sv], rv, gsem)

        def step(i, r8):
            sv, dv, sem = idx[r8]
            rv, gsem = rows[r8 % 4]
            svn, dvn, semn = idx[(r8 + 3) % 8]
            rvn, gsemn = rows[(r8 + 3) % 4]
            pltpu.make_async_copy(g_hbm.at[sv], rv, gsem).wait()

            @pl.when(i < STEPS - 3)
            def _():
                wait_idx(i + 3, svn, dvn, semn)
                pltpu.async_copy(g_hbm.at[svn], rvn, gsemn)

            pltpu.sync_copy(rv, acc.at[dv], add=True)

            @pl.when(i < STEPS - 8)
            def _():
                load_idx(i + 8, sv, dv, sem)

        def body(i, carry):
            for r8 in range(8):
                @pl.when(lax.rem(i, 8) == r8)
                def _(r8=r8):
                    step(i, r8)
            return carry

        lax.fori_loop(0, STEPS, body, 0)

        plsc.subcore_barrier()
        pltpu.sync_copy(acc.at[pl.ds(s * RPT, RPT)],
                        out_hbm.at[c, pl.ds(s * RPT, RPT)])

    return k


def _deg_pass():
    mesh = plsc.VectorSubcoreMesh(core_axis_name="c", subcore_axis_name="s")

    @functools.partial(
        pl.kernel,
        mesh=mesh,
        out_type=jax.ShapeDtypeStruct((NC, NPAD, DW), jnp.float32),
        scratch_types=(
            [pltpu.VMEM((K,), jnp.int32)] * 8
            + [pltpu.VMEM((K, DW), jnp.float32)]
            + [pltpu.VMEM_SHARED((NPAD, DW), jnp.float32)]
            + [pltpu.SemaphoreType.DMA] * 8
        ),
    )
    def k(dst_hbm, ones_hbm, zeros_hbm, out_hbm,
          d0, d1, d2, d3, d4, d5, d6, d7, ones_v, acc,
          i0, i1, i2, i3, i4, i5, i6, i7):
        c = lax.axis_index("c")
        s = lax.axis_index("s")
        tid = c * NS + s
        ebase = pl.multiple_of(tid * EPT, 16)
        pltpu.sync_copy(zeros_hbm, acc.at[pl.ds(s * RPT, RPT)])
        pltpu.sync_copy(ones_hbm, ones_v)
        plsc.subcore_barrier()

        def eslice(i):
            return pl.ds(pl.multiple_of(ebase + i * K, 16), K)

        idx = [(d0, i0), (d1, i1), (d2, i2), (d3, i3),
               (d4, i4), (d5, i5), (d6, i6), (d7, i7)]

        for cch in range(8):
            dv, sem = idx[cch]
            pltpu.async_copy(dst_hbm.at[eslice(cch)], dv, sem)

        def step(i, r8):
            dv, sem = idx[r8]
            pltpu.make_async_copy(dst_hbm.at[eslice(i)], dv, sem).wait()
            pltpu.sync_copy(ones_v, acc.at[dv], add=True)

            @pl.when(i < STEPS - 8)
            def _():
                pltpu.async_copy(dst_hbm.at[eslice(i + 8)], dv, sem)

        def body(i, carry):
            for r8 in range(8):
                @pl.when(lax.rem(i, 8) == r8)
                def _(r8=r8):
                    step(i, r8)
            return carry

        lax.fori_loop(0, STEPS, body, 0)
        plsc.subcore_barrier()
        pltpu.sync_copy(acc.at[pl.ds(s * RPT, RPT)],
                        out_hbm.at[c, pl.ds(s * RPT, RPT)])

    return k


def _dinv_from(dp_ref):
    deg = dp_ref[0, :, 0] + dp_ref[1, :, 0] + 1.0
    return lax.rsqrt(deg)[:, None]


BS = 1024
GRID = NPAD // BS


def _stage1(x, W1, dp):
    def body(x_ref, w_ref, dp_ref, g_ref):
        h = jnp.dot(x_ref[...], w_ref[...], preferred_element_type=jnp.float32)
        g_ref[...] = _dinv_from(dp_ref) * h

    return pl.pallas_call(
        body,
        grid=(GRID,),
        in_specs=[
            pl.BlockSpec((BS, 128), lambda i: (i, 0)),
            pl.BlockSpec((128, 128), lambda i: (0, 0)),
            pl.BlockSpec((2, BS, DW), lambda i: (0, i, 0)),
        ],
        out_specs=pl.BlockSpec((BS, 128), lambda i: (i, 0)),
        out_shape=jax.ShapeDtypeStruct((NPAD, 128), jnp.float32),
    )(x, W1, dp)


def _stage2(s1, g1, dp, b1, W2p):
    def body(s_ref, g_ref, dp_ref, b_ref, w_ref, o_ref):
        dinv = _dinv_from(dp_ref)
        pre = dinv * (s_ref[0] + s_ref[1] + g_ref[...]) + b_ref[...][None, :]
        a = jnp.maximum(pre, 0.0)
        o_ref[...] = dinv * jnp.dot(a, w_ref[...],
                                    preferred_element_type=jnp.float32)

    return pl.pallas_call(
        body,
        grid=(GRID,),
        in_specs=[
            pl.BlockSpec((2, BS, 128), lambda i: (0, i, 0)),
            pl.BlockSpec((BS, 128), lambda i: (i, 0)),
            pl.BlockSpec((2, BS, DW), lambda i: (0, i, 0)),
            pl.BlockSpec((128,), lambda i: (0,)),
            pl.BlockSpec((128, 128), lambda i: (0, 0)),
        ],
        out_specs=pl.BlockSpec((BS, 128), lambda i: (i, 0)),
        out_shape=jax.ShapeDtypeStruct((NPAD, 128), jnp.float32),
    )(s1, g1, dp, b1, W2p)


def _stage3(s2, g2, dp, b2p):
    def body(s_ref, g_ref, dp_ref, b_ref, o_ref):
        dinv = _dinv_from(dp_ref)
        pre = dinv * (s_ref[0] + s_ref[1] + g_ref[...]) + b_ref[...][None, :]
        o_ref[...] = jnp.maximum(pre[:, :64], 0.0)

    return pl.pallas_call(
        body,
        grid=(GRID,),
        in_specs=[
            pl.BlockSpec((2, BS, 128), lambda i: (0, i, 0)),
            pl.BlockSpec((BS, 128), lambda i: (i, 0)),
            pl.BlockSpec((2, BS, DW), lambda i: (0, i, 0)),
            pl.BlockSpec((128,), lambda i: (0,)),
        ],
        out_specs=pl.BlockSpec((BS, 64), lambda i: (i, 0)),
        out_shape=jax.ShapeDtypeStruct((NPAD, 64), jnp.float32),
    )(s2, g2, dp, b2p)


def kernel(x, ei, W1, b1, W2, b2):
    src = ei[0]
    dst = ei[1]
    xp = jnp.zeros((NPAD, 128), jnp.float32).at[:N].set(x)

    zeros128 = jnp.zeros((RPT, 128), jnp.float32)
    ones_dw = jnp.ones((K, DW), jnp.float32)
    dp = _deg_pass()(dst, ones_dw, zeros128)

    W2p = jnp.zeros((128, 128), jnp.float32).at[:, :64].set(W2)
    b2p = jnp.zeros((128,), jnp.float32).at[:64].set(b2)

    g1 = _stage1(xp, W1, dp)
    s1 = _edge_pass(128)(g1, src, dst, zeros128)
    g2 = _stage2(s1, g1, dp, b1, W2p)
    s2 = _edge_pass(128)(g2, src, dst, zeros128)
    out = _stage3(s2, g2, dp, b2p)
    return out[:N]


# speedup vs baseline: 30.3702x; 1.0023x over previous
import functools

import jax
import jax.numpy as jnp
from jax import lax
from jax.experimental import pallas as pl
from jax.experimental.pallas import tpu as pltpu
from jax.experimental.pallas import tpu_sc as plsc

N = 10000
NPAD = 10240
E = 320000
NC = 2
NS = 16
NT = NC * NS
EPT = E // NT
K = 80
STEPS = EPT // K
HALF = (STEPS - 1) // 2
RPT = NPAD // NS
DW = 128


def _edge_pass(width):
    mesh = plsc.VectorSubcoreMesh(core_axis_name="c", subcore_axis_name="s")

    @functools.partial(
        pl.kernel,
        mesh=mesh,
        out_type=jax.ShapeDtypeStruct((NC, NPAD, width), jnp.float32),
        scratch_types=(
            [pltpu.VMEM((K,), jnp.int32)] * 16
            + [pltpu.VMEM((K, width), jnp.float32)] * 4
            + [pltpu.VMEM_SHARED((NPAD, width), jnp.float32)]
            + [pltpu.SemaphoreType.DMA] * 12
        ),
    )
    def k(g_hbm, src_hbm, dst_hbm, zeros_hbm, out_hbm,
          s0, d0, s1, d1, s2, d2, s3, d3, s4, d4, s5, d5, s6, d6, s7, d7,
          r0, r1, r2, r3, acc,
          g0, g1, g2, g3, i0, i1, i2, i3, i4, i5, i6, i7):
        c = lax.axis_index("c")
        s = lax.axis_index("s")
        tid = c * NS + s
        ebase = pl.multiple_of(tid * EPT, 16)
        pltpu.sync_copy(zeros_hbm, acc.at[pl.ds(s * RPT, RPT)])
        plsc.subcore_barrier()

        def eslice(i):
            return pl.ds(pl.multiple_of(ebase + i * K, 16), K)

        def load_idx(i, sv, dv, sem):
            pltpu.async_copy(src_hbm.at[eslice(i)], sv, sem)
            pltpu.async_copy(dst_hbm.at[eslice(i)], dv, sem)

        def wait_idx(i, sv, dv, sem):
            pltpu.make_async_copy(src_hbm.at[eslice(i)], sv, sem).wait()
            pltpu.make_async_copy(dst_hbm.at[eslice(i)], dv, sem).wait()

        idx = [(s0, d0, i0), (s1, d1, i1), (s2, d2, i2), (s3, d3, i3),
               (s4, d4, i4), (s5, d5, i5), (s6, d6, i6), (s7, d7, i7)]
        rows = [(r0, g0), (r1, g1), (r2, g2), (r3, g3)]

        for cch in range(8):
            sv, dv, sem = idx[cch]
            load_idx(cch, sv, dv, sem)
        for cch in range(4):
            sv, dv, sem = idx[cch]
            rv, gsem = rows[cch]
            wait_idx(cch, sv, dv, sem)
            pltpu.async_copy(g_hbm.at[sv], rv, gsem)

        def step(i, r8):
            sv, dv, sem = idx[r8]
            rv, gsem = rows[r8 % 4]
            svn, dvn, semn = idx[(r8 + 4) % 8]
            pltpu.make_async_copy(g_hbm.at[sv], rv, gsem).wait()
            pltpu.sync_copy(rv, acc.at[dv], add=True)

            @pl.when(i < STEPS - 4)
            def _():
                wait_idx(i + 4, svn, dvn, semn)
                pltpu.async_copy(g_hbm.at[svn], rv, gsem)

            @pl.when(i < STEPS - 8)
            def _():
                load_idx(i + 8, sv, dv, sem)

        def body(i, carry):
            for r8 in range(8):
                @pl.when(lax.rem(i, 8) == r8)
                def _(r8=r8):
                    step(i, r8)
            return carry

        lax.fori_loop(0, STEPS, body, 0)

        plsc.subcore_barrier()
        pltpu.sync_copy(acc.at[pl.ds(s * RPT, RPT)],
                        out_hbm.at[c, pl.ds(s * RPT, RPT)])

    return k


def _deg_pass():
    mesh = plsc.VectorSubcoreMesh(core_axis_name="c", subcore_axis_name="s")

    @functools.partial(
        pl.kernel,
        mesh=mesh,
        out_type=jax.ShapeDtypeStruct((NC, NPAD, DW), jnp.float32),
        scratch_types=(
            [pltpu.VMEM((K,), jnp.int32)] * 8
            + [pltpu.VMEM((K, DW), jnp.float32)]
            + [pltpu.VMEM_SHARED((NPAD, DW), jnp.float32)]
            + [pltpu.SemaphoreType.DMA] * 8
        ),
    )
    def k(dst_hbm, ones_hbm, zeros_hbm, out_hbm,
          d0, d1, d2, d3, d4, d5, d6, d7, ones_v, acc,
          i0, i1, i2, i3, i4, i5, i6, i7):
        c = lax.axis_index("c")
        s = lax.axis_index("s")
        tid = c * NS + s
        ebase = pl.multiple_of(tid * EPT, 16)
        pltpu.sync_copy(zeros_hbm, acc.at[pl.ds(s * RPT, RPT)])
        pltpu.sync_copy(ones_hbm, ones_v)
        plsc.subcore_barrier()

        def eslice(i):
            return pl.ds(pl.multiple_of(ebase + i * K, 16), K)

        idx = [(d0, i0), (d1, i1), (d2, i2), (d3, i3),
               (d4, i4), (d5, i5), (d6, i6), (d7, i7)]

        for cch in range(8):
            dv, sem = idx[cch]
            pltpu.async_copy(dst_hbm.at[eslice(cch)], dv, sem)

        def step(i, r8):
            dv, sem = idx[r8]
            pltpu.make_async_copy(dst_hbm.at[eslice(i)], dv, sem).wait()
            pltpu.sync_copy(ones_v, acc.at[dv], add=True)

            @pl.when(i < STEPS - 8)
            def _():
                pltpu.async_copy(dst_hbm.at[eslice(i + 8)], dv, sem)

        def body(i, carry):
            for r8 in range(8):
                @pl.when(lax.rem(i, 8) == r8)
                def _(r8=r8):
                    step(i, r8)
            return carry

        lax.fori_loop(0, STEPS, body, 0)
        plsc.subcore_barrier()
        pltpu.sync_copy(acc.at[pl.ds(s * RPT, RPT)],
                        out_hbm.at[c, pl.ds(s * RPT, RPT)])

    return k


def _dinv_from(dp_ref):
    deg = dp_ref[0, :, 0] + dp_ref[1, :, 0] + 1.0
    return lax.rsqrt(deg)[:, None]


BS = 1024
GRID = NPAD // BS


def _stage1(x, W1, dp):
    def body(x_ref, w_ref, dp_ref, g_ref, dinv_ref):
        dinv = _dinv_from(dp_ref)
        h = jnp.dot(x_ref[...], w_ref[...], preferred_element_type=jnp.float32)
        g_ref[...] = dinv * h
        dinv_ref[...] = jnp.broadcast_to(dinv, (BS, 128))

    return pl.pallas_call(
        body,
        grid=(GRID,),
        in_specs=[
            pl.BlockSpec((BS, 128), lambda i: (i, 0)),
            pl.BlockSpec((128, 128), lambda i: (0, 0)),
            pl.BlockSpec((2, BS, DW), lambda i: (0, i, 0)),
        ],
        out_specs=[
            pl.BlockSpec((BS, 128), lambda i: (i, 0)),
            pl.BlockSpec((BS, 128), lambda i: (i, 0)),
        ],
        out_shape=[
            jax.ShapeDtypeStruct((NPAD, 128), jnp.float32),
            jax.ShapeDtypeStruct((NPAD, 128), jnp.float32),
        ],
    )(x, W1, dp)


def _stage2(s1, g1, dinv, b1, W2p):
    def body(s_ref, g_ref, dinv_ref, b_ref, w_ref, o_ref):
        dinv_b = dinv_ref[...]
        pre = dinv_b * (s_ref[0] + s_ref[1] + g_ref[...]) + b_ref[...][None, :]
        a = jnp.maximum(pre, 0.0)
        o_ref[...] = dinv_b * jnp.dot(a, w_ref[...],
                                      preferred_element_type=jnp.float32)

    return pl.pallas_call(
        body,
        grid=(GRID,),
        in_specs=[
            pl.BlockSpec((2, BS, 128), lambda i: (0, i, 0)),
            pl.BlockSpec((BS, 128), lambda i: (i, 0)),
            pl.BlockSpec((BS, 128), lambda i: (i, 0)),
            pl.BlockSpec((128,), lambda i: (0,)),
            pl.BlockSpec((128, 128), lambda i: (0, 0)),
        ],
        out_specs=pl.BlockSpec((BS, 128), lambda i: (i, 0)),
        out_shape=jax.ShapeDtypeStruct((NPAD, 128), jnp.float32),
    )(s1, g1, dinv, b1, W2p)


def _stage3(s2, g2, dinv, b2p):
    def body(s_ref, g_ref, dinv_ref, b_ref, o_ref):
        pre = (dinv_ref[...] * (s_ref[0] + s_ref[1] + g_ref[...])
               + b_ref[...][None, :])
        o_ref[...] = jnp.maximum(pre[:, :64], 0.0)

    return pl.pallas_call(
        body,
        grid=(GRID,),
        in_specs=[
            pl.BlockSpec((2, BS, 128), lambda i: (0, i, 0)),
            pl.BlockSpec((BS, 128), lambda i: (i, 0)),
            pl.BlockSpec((BS, 128), lambda i: (i, 0)),
            pl.BlockSpec((128,), lambda i: (0,)),
        ],
        out_specs=pl.BlockSpec((BS, 64), lambda i: (i, 0)),
        out_shape=jax.ShapeDtypeStruct((N, 64), jnp.float32),
    )(s2, g2, dinv, b2p)


def kernel(x, ei, W1, b1, W2, b2):
    src = ei[0]
    dst = ei[1]
    xp = jnp.zeros((NPAD, 128), jnp.float32).at[:N].set(x)

    zeros128 = jnp.zeros((RPT, 128), jnp.float32)
    ones_dw = jnp.ones((K, DW), jnp.float32)
    dp = _deg_pass()(dst, ones_dw, zeros128)

    W2p = jnp.zeros((128, 128), jnp.float32).at[:, :64].set(W2)
    b2p = jnp.zeros((128,), jnp.float32).at[:64].set(b2)

    g1, dinv = _stage1(xp, W1, dp)
    s1 = _edge_pass(128)(g1, src, dst, zeros128)
    g2 = _stage2(s1, g1, dinv, b1, W2p)
    s2 = _edge_pass(128)(g2, src, dst, zeros128)
    return _stage3(s2, g2, dinv, b2p)
